# baseline (speedup 1.0000x reference)
import jax
import jax.numpy as jnp
from jax import lax
from jax.experimental import pallas as pl
from jax.experimental.pallas import tpu as pltpu
from jax.experimental.pallas import tpu_sc as plsc

N = 10000
E = 320000
NP = 10240
D = 128
HD = 64
BLK_A = 1024
BLK_C = 2048
EDGES_PER_TILE_B = E // 16
EDGES_PER_TILE_D = E // 32
CH = 2000
NB = CH // 16


def _proj_body(x_ref, w_ref, wp_ref, a_ref, h_ref, hb_ref, aa_ref):
    xb = x_ref[...]
    h = jnp.dot(xb, w_ref[0], preferred_element_type=jnp.float32)
    h_ref[...] = h
    hp = jnp.dot(xb, wp_ref[0], preferred_element_type=jnp.float32)
    hb_ref[...] = hp.astype(jnp.bfloat16)
    aa_ref[0] = lax.dot_general(a_ref[0], h, (((0,), (1,)), ((), ())),
                                preferred_element_type=jnp.float32)


def _proj(x_p, W2, W2p, A2):
    return pl.pallas_call(
        _proj_body,
        grid=(NP // BLK_A, 2),
        in_specs=[
            pl.BlockSpec((BLK_A, D), lambda i, j: (i, 0)),
            pl.BlockSpec((1, D, D), lambda i, j: (j, 0, 0)),
            pl.BlockSpec((1, D, D), lambda i, j: (j, 0, 0)),
            pl.BlockSpec((1, D, 8), lambda i, j: (j, 0, 0)),
        ],
        out_specs=[
            pl.BlockSpec((BLK_A, D), lambda i, j: (i + 10 * j, 0)),
            pl.BlockSpec((BLK_A, D), lambda i, j: (i + 10 * j, 0)),
            pl.BlockSpec((1, 8, BLK_A), lambda i, j: (j, 0, i)),
        ],
        out_shape=[
            jax.ShapeDtypeStruct((2 * NP, D), jnp.float32),
            jax.ShapeDtypeStruct((2 * NP, D), jnp.bfloat16),
            jax.ShapeDtypeStruct((2, 8, NP), jnp.float32),
        ],
    )(x_p, W2, W2p, A2)


def _logit_body(tbl2, src_h, dst_h, zden, den_out, w0_out, w1_out,
                tbl_v, den_v, srcb, dstb, w0b, w1b):
    c = lax.axis_index("c")
    t = lax.axis_index("s")
    pltpu.sync_copy(tbl2.at[pl.ds(c * 4 * NP, 4 * NP)], tbl_v)
    pltpu.sync_copy(zden, den_v)
    tbase = t * EDGES_PER_TILE_B
    ebase = c * E + tbase
    fwd = c == 0

    @pl.loop(0, EDGES_PER_TILE_B // CH)
    def _chunk(ci):
        pltpu.sync_copy(src_h.at[pl.ds(tbase + ci * CH, CH)], srcb)
        pltpu.sync_copy(dst_h.at[pl.ds(tbase + ci * CH, CH)], dstb)

        @pl.loop(0, NB)
        def _batch(bi):
            off = bi * 16
            sa = srcb[pl.ds(off, 16)]
            da = dstb[pl.ds(off, 16)]
            s = jnp.where(fwd, sa, da)
            d = jnp.where(fwd, da, sa)
            as0 = plsc.load_gather(tbl_v, [s])
            as1 = plsc.load_gather(tbl_v, [s + NP])
            ad0 = plsc.load_gather(tbl_v, [d + 2 * NP])
            ad1 = plsc.load_gather(tbl_v, [d + 3 * NP])
            e0 = as0 + ad0
            e1 = as1 + ad1
            e0 = jnp.where(e0 >= 0, e0, 0.2 * e0)
            e1 = jnp.where(e1 >= 0, e1, 0.2 * e1)
            w0 = jnp.exp(e0)
            w1 = jnp.exp(e1)
            w0b[pl.ds(off, 16)] = w0
            w1b[pl.ds(off, 16)] = w1
            plsc.addupdate_scatter(den_v, [d], w0)
            plsc.addupdate_scatter(den_v, [d + NP], w1)

        pltpu.sync_copy(w0b, w0_out.at[pl.ds(ebase + ci * CH, CH)])
        pltpu.sync_copy(w1b, w1_out.at[pl.ds(ebase + ci * CH, CH)])

    pltpu.sync_copy(den_v, den_out.at[c * 16 + t])


def _logits(tbl2, src, dst, zden):
    mesh = plsc.VectorSubcoreMesh(core_axis_name="c", subcore_axis_name="s")
    f = pl.kernel(
        _logit_body,
        out_type=(jax.ShapeDtypeStruct((32, 2 * NP), jnp.float32),
                  jax.ShapeDtypeStruct((2 * E,), jnp.float32),
                  jax.ShapeDtypeStruct((2 * E,), jnp.float32)),
        mesh=mesh,
        compiler_params=pltpu.CompilerParams(needs_layout_passes=False),
        scratch_types=[
            pltpu.VMEM((4 * NP,), jnp.float32),
            pltpu.VMEM((2 * NP,), jnp.float32),
            pltpu.VMEM((CH,), jnp.int32),
            pltpu.VMEM((CH,), jnp.int32),
            pltpu.VMEM((CH,), jnp.float32),
            pltpu.VMEM((CH,), jnp.float32),
        ],
    )
    return f(tbl2, src, dst, zden)


NBUF = 10
CH_B2 = 800
NB_B2 = CH_B2 // 16


def _scat_body(h2b, src_h, dst_h, w0_in, w1_in, z2, acc_out, acc_sh,
               srcb, dstb, w0b, w1b, gbufs, rowbufs, semg, sems):
    c = lax.axis_index("c")
    t = lax.axis_index("s")
    pltpu.sync_copy(z2, acc_sh.at[pl.ds(t * (NP // 16), NP // 16)])
    plsc.subcore_barrier()

    tbase = t * EDGES_PER_TILE_B
    ebase = c * E + tbase
    coff = c * NP
    fwd = c == 0
    himask = jnp.full((16,), -65536, jnp.int32)

    def issue_gather(m, b):
        sa = srcb[pl.ds(m * 16, 16)]
        da = dstb[pl.ds(m * 16, 16)]
        s = jnp.where(fwd, sa, da)
        return pltpu.async_copy(h2b.at[s + coff], gbufs[b], semg[b])

    @pl.loop(0, EDGES_PER_TILE_B // CH_B2)
    def _chunk(ci):
        pltpu.sync_copy(src_h.at[pl.ds(tbase + ci * CH_B2, CH_B2)], srcb)
        pltpu.sync_copy(dst_h.at[pl.ds(tbase + ci * CH_B2, CH_B2)], dstb)
        pltpu.sync_copy(w0_in.at[pl.ds(ebase + ci * CH_B2, CH_B2)], w0b)
        pltpu.sync_copy(w1_in.at[pl.ds(ebase + ci * CH_B2, CH_B2)], w1b)
        for b in range(NBUF):
            issue_gather(b, b)

        @pl.loop(0, NB_B2, step=NBUF)
        def _batch(bi):
            for b in range(NBUF):
                m = bi + b
                off = m * 16
                sa = dstb[pl.ds(off, 16)]
                sb = srcb[pl.ds(off, 16)]
                d = jnp.where(fwd, sa, sb)
                w0 = w0b[pl.ds(off, 16)]
                w1 = w1b[pl.ds(off, 16)]
                pltpu.make_async_copy(h2b.at[pl.ds(0, 16)], gbufs[b],
                                      semg[b]).wait()
                @pl.when(m >= NBUF)
                def _():
                    pltpu.make_async_copy(h2b.at[pl.ds(0, 16)], rowbufs[b],
                                          sems[b]).wait()
                gbuf = gbufs[b]
                rowbuf = rowbufs[b]
                for r in range(16):
                    w0r = w0[r]
                    w1r = w1[r]
                    for u in range(4):
                        v = gbuf[r, pl.ds(32 * u, 32)]
                        x = plsc.bitcast(v, jnp.int32)
                        lo = plsc.bitcast(x << 16, jnp.float32)
                        hi = plsc.bitcast(x & himask, jnp.float32)
                        rowbuf[r, pl.ds(16 * u, 16)] = lo * w0r
                        rowbuf[r, pl.ds(HD + 16 * u, 16)] = hi * w1r
                pltpu.async_copy(rowbufs[b], acc_sh.at[d], sems[b], add=True)

                @pl.when(m + NBUF < NB_B2)
                def _():
                    issue_gather(m + NBUF, b)

        for b in range(NBUF):
            pltpu.make_async_copy(h2b.at[pl.ds(0, 16)], rowbufs[b],
                                  sems[b]).wait()

    plsc.subcore_barrier()
    pltpu.sync_copy(acc_sh.at[pl.ds(t * (NP // 16), NP // 16)],
                    acc_out.at[c, pl.ds(t * (NP // 16), NP // 16)])


def _scatter(h2b, src, dst, w0, w1, z2):
    mesh = plsc.VectorSubcoreMesh(core_axis_name="c", subcore_axis_name="s")
    f = pl.kernel(
        _scat_body,
        out_type=jax.ShapeDtypeStruct((2, NP, D), jnp.float32),
        mesh=mesh,
        compiler_params=pltpu.CompilerParams(needs_layout_passes=False,
                                             use_tc_tiling_on_sc=False),
        scratch_types=[
            pltpu.VMEM_SHARED((NP, D), jnp.float32),
            pltpu.VMEM((CH_B2,), jnp.int32),
            pltpu.VMEM((CH_B2,), jnp.int32),
            pltpu.VMEM((CH_B2,), jnp.float32),
            pltpu.VMEM((CH_B2,), jnp.float32),
            [pltpu.VMEM((16, D), jnp.bfloat16)] * NBUF,
            [pltpu.VMEM((16, D), jnp.float32)] * NBUF,
            [pltpu.SemaphoreType.DMA] * NBUF,
            [pltpu.SemaphoreType.DMA] * NBUF,
        ],
    )
    return f(h2b, src, dst, w0, w1, z2)


def _mid_body(num_ref, den_ref, aa_ref, hf_ref, hu_ref, b2_ref, wfc_ref,
              bfc_ref, wo_ref, h3_ref):
    num = num_ref[...]
    den = jnp.sum(den_ref[...], axis=1)
    aa = aa_ref[...]

    def gat_out(k, h):
        el0 = aa[k, 0] + aa[k, 2]
        el1 = aa[k, 1] + aa[k, 3]
        wl0 = jnp.exp(jnp.where(el0 >= 0, el0, 0.2 * el0))
        wl1 = jnp.exp(jnp.where(el1 >= 0, el1, 0.2 * el1))
        n0 = num[k, :, :HD] + wl0[:, None] * h[:, :HD]
        n1 = num[k, :, HD:] + wl1[:, None] * h[:, HD:]
        d0 = den[k, 0] + wl0 + 1e-16
        d1 = den[k, 1] + wl1 + 1e-16
        out = jnp.concatenate([n0 / d0[:, None], n1 / d1[:, None]], axis=1)
        return jnp.maximum(out + b2_ref[k][None, :], 0.0)

    Hf = gat_out(0, hf_ref[...])
    Hu = gat_out(1, hu_ref[...])
    hcat = jnp.concatenate([Hf, Hu], axis=1)
    h = jnp.dot(hcat, wfc_ref[...], preferred_element_type=jnp.float32)
    h = jnp.maximum(h + bfc_ref[...], 0.0)
    h3_ref[...] = jnp.sum(h * wo_ref[...], axis=1)


def _mid(num2, den2, aa2, h2, b2, W_fc, bfc, wo_row):
    nblk = NP // BLK_C
    return pl.pallas_call(
        _mid_body,
        grid=(nblk,),
        in_specs=[
            pl.BlockSpec((2, BLK_C, D), lambda i: (0, i, 0)),
            pl.BlockSpec((2, 16, 2, BLK_C), lambda i: (0, 0, 0, i)),
            pl.BlockSpec((2, 8, BLK_C), lambda i: (0, 0, i)),
            pl.BlockSpec((BLK_C, D), lambda i: (i, 0)),
            pl.BlockSpec((BLK_C, D), lambda i: (i + nblk, 0)),
            pl.BlockSpec((2, D), lambda i: (0, 0)),
            pl.BlockSpec((2 * D, D), lambda i: (0, 0)),
            pl.BlockSpec((1, D), lambda i: (0, 0)),
            pl.BlockSpec((1, D), lambda i: (0, 0)),
        ],
        out_specs=pl.BlockSpec((BLK_C,), lambda i: (i,)),
        out_shape=jax.ShapeDtypeStruct((NP,), jnp.float32),
    )(num2, den2, aa2, h2, h2, b2, W_fc, bfc, wo_row)


def _out_body(h3, src, dst, zden, cvec, num_out, den_out, h3_v, num_v,
              den_v, srcb, dstb, cv, sem):
    c = lax.axis_index("c")
    t = lax.axis_index("s")
    pltpu.sync_copy(h3, h3_v)
    pltpu.sync_copy(zden.at[pl.ds(0, NP)], num_v)
    pltpu.sync_copy(zden.at[pl.ds(0, NP)], den_v)
    pltpu.sync_copy(cvec, cv)
    cvv = cv[...]
    cs = cvv[0]
    cd = cvv[1]
    ebase = (c * 16 + t) * EDGES_PER_TILE_D

    @pl.loop(0, EDGES_PER_TILE_D // CH)
    def _chunk(ci):
        pltpu.sync_copy(src.at[pl.ds(ebase + ci * CH, CH)], srcb)
        pltpu.sync_copy(dst.at[pl.ds(ebase + ci * CH, CH)], dstb)

        @pl.loop(0, NB)
        def _batch(bi):
            off = bi * 16
            s = srcb[pl.ds(off, 16)]
            d = dstb[pl.ds(off, 16)]
            hs = plsc.load_gather(h3_v, [s])
            hd = plsc.load_gather(h3_v, [d])
            e = cs * hs + cd * hd
            e = jnp.where(e >= 0, e, 0.2 * e)
            w = jnp.exp(e)
            plsc.addupdate_scatter(num_v, [d], w * hs)
            plsc.addupdate_scatter(den_v, [d], w)

    pltpu.sync_copy(num_v, num_out.at[c * 16 + t])
    pltpu.sync_copy(den_v, den_out.at[c * 16 + t])


def _conv_o(h3, src, dst, zden, cvec):
    mesh = plsc.VectorSubcoreMesh(core_axis_name="c", subcore_axis_name="s")
    f = pl.kernel(
        _out_body,
        out_type=(jax.ShapeDtypeStruct((32, NP), jnp.float32),
                  jax.ShapeDtypeStruct((32, NP), jnp.float32)),
        mesh=mesh,
        compiler_params=pltpu.CompilerParams(needs_layout_passes=False),
        scratch_types=[
            pltpu.VMEM((NP,), jnp.float32),
            pltpu.VMEM((NP,), jnp.float32),
            pltpu.VMEM((NP,), jnp.float32),
            pltpu.VMEM((CH,), jnp.int32),
            pltpu.VMEM((CH,), jnp.int32),
            pltpu.VMEM((16,), jnp.float32),
            pltpu.SemaphoreType.DMA,
        ],
    )
    return f(h3, src, dst, zden, cvec)


def _fin_body(h3_ref, num_ref, den_ref, scal_ref, out_ref):
    h3 = h3_ref[...]
    cs = scal_ref[0, 0]
    cd = scal_ref[0, 1]
    bo = scal_ref[0, 2]
    el = (cs + cd) * h3
    wl = jnp.exp(jnp.where(el >= 0, el, 0.2 * el))
    num = jnp.sum(num_ref[...], axis=0) + wl * h3
    den = jnp.sum(den_ref[...], axis=0) + wl + 1e-16
    out_ref[...] = jax.nn.sigmoid(num / den + bo)


def _fin(h3, num3, den3, scal):
    return pl.pallas_call(
        _fin_body,
        grid=(1,),
        in_specs=[
            pl.BlockSpec((NP,), lambda i: (0,)),
            pl.BlockSpec((32, NP), lambda i: (0, 0)),
            pl.BlockSpec((32, NP), lambda i: (0, 0)),
            pl.BlockSpec((1, 128), lambda i: (0, 0)),
        ],
        out_specs=pl.BlockSpec((NP,), lambda i: (0,)),
        out_shape=jax.ShapeDtypeStruct((NP,), jnp.float32),
    )(h3, num3, den3, scal)


def _pack_att(att_src, att_dst):
    a = jnp.zeros((D, 8), jnp.float32)
    a = a.at[:HD, 0].set(att_src[0])
    a = a.at[HD:, 1].set(att_src[1])
    a = a.at[:HD, 2].set(att_dst[0])
    a = a.at[HD:, 3].set(att_dst[1])
    return a


@jax.jit
def kernel(x, edge_index, W_f, att_src_f, att_dst_f, b_f,
           W_u, att_src_u, att_dst_u, b_u,
           W_fc, b_fc, W_o, att_src_o, att_dst_o, b_o):
    src = edge_index[0].astype(jnp.int32)
    dst = edge_index[1].astype(jnp.int32)
    x_p = jnp.pad(x, ((0, NP - N), (0, 0)))

    W2 = jnp.stack([W_f, W_u])
    perm = jnp.arange(D).reshape(2, HD).T.reshape(-1)
    W2p = W2[:, :, perm]
    A2 = jnp.stack([_pack_att(att_src_f, att_dst_f),
                    _pack_att(att_src_u, att_dst_u)])
    h2, h2b, aa2 = _proj(x_p, W2, W2p, A2)

    tbl2 = aa2[:, :4, :].reshape(-1)
    z2 = jnp.zeros((NP // 16, D), jnp.float32)
    zden = jnp.zeros((2 * NP,), jnp.float32)
    den, w0, w1 = _logits(tbl2, src, dst, zden)
    acc = _scatter(h2b, src, dst, w0, w1, z2)

    den2 = den.reshape(2, 16, 2, NP)
    b2 = jnp.stack([b_f, b_u])
    h3 = _mid(acc, den2, aa2, h2, b2, W_fc, b_fc.reshape(1, D),
              W_o.reshape(1, D))

    cvec = jnp.zeros((16,), jnp.float32)
    cvec = cvec.at[0].set(att_src_o[0, 0]).at[1].set(att_dst_o[0, 0])
    num3, den3 = _conv_o(h3, src, dst, zden, cvec)

    scal = jnp.zeros((1, 128), jnp.float32)
    scal = scal.at[0, 0].set(att_src_o[0, 0])
    scal = scal.at[0, 1].set(att_dst_o[0, 0])
    scal = scal.at[0, 2].set(b_o[0])
    num3, den3 = num3, den3
    out = _fin(h3, num3, den3, scal)
    return out[:N, None]


# speedup vs baseline: 1.3416x; 1.3416x over previous
import jax
import jax.numpy as jnp
from jax import lax
from jax.experimental import pallas as pl
from jax.experimental.pallas import tpu as pltpu
from jax.experimental.pallas import tpu_sc as plsc

N = 10000
E = 320000
NP = 10240
D = 128
HD = 64
BLK_A = 1024
BLK_C = 2048
EDGES_PER_TILE_B = E // 16
EDGES_PER_TILE_D = E // 32
CH = 2000
NB = CH // 16


def _proj_body(x_ref, w_ref, wp_ref, a_ref, h_ref, hb_ref, aa_ref):
    xb = x_ref[...]
    h = jnp.dot(xb, w_ref[0], preferred_element_type=jnp.float32)
    h_ref[...] = h
    hp = jnp.dot(xb, wp_ref[0], preferred_element_type=jnp.float32)
    hb_ref[...] = hp.astype(jnp.bfloat16)
    aa_ref[0] = lax.dot_general(a_ref[0], h, (((0,), (1,)), ((), ())),
                                preferred_element_type=jnp.float32)


def _proj(x_p, W2, W2p, A2):
    return pl.pallas_call(
        _proj_body,
        grid=(NP // BLK_A, 2),
        in_specs=[
            pl.BlockSpec((BLK_A, D), lambda i, j: (i, 0)),
            pl.BlockSpec((1, D, D), lambda i, j: (j, 0, 0)),
            pl.BlockSpec((1, D, D), lambda i, j: (j, 0, 0)),
            pl.BlockSpec((1, D, 8), lambda i, j: (j, 0, 0)),
        ],
        out_specs=[
            pl.BlockSpec((BLK_A, D), lambda i, j: (i + 10 * j, 0)),
            pl.BlockSpec((BLK_A, D), lambda i, j: (i + 10 * j, 0)),
            pl.BlockSpec((1, 8, BLK_A), lambda i, j: (j, 0, i)),
        ],
        out_shape=[
            jax.ShapeDtypeStruct((2 * NP, D), jnp.float32),
            jax.ShapeDtypeStruct((2 * NP, D), jnp.bfloat16),
            jax.ShapeDtypeStruct((2, 8, NP), jnp.float32),
        ],
    )(x_p, W2, W2p, A2)


def _logit_body(tbl2, src_h, dst_h, zden, den_out, w0_out, w1_out,
                tbl_v, den_v, srcb, dstb, w0b, w1b):
    c = lax.axis_index("c")
    t = lax.axis_index("s")
    pltpu.sync_copy(tbl2.at[pl.ds(c * 4 * NP, 4 * NP)], tbl_v)
    pltpu.sync_copy(zden, den_v)
    tbase = t * EDGES_PER_TILE_B
    ebase = c * E + tbase
    fwd = c == 0

    @pl.loop(0, EDGES_PER_TILE_B // CH)
    def _chunk(ci):
        pltpu.sync_copy(src_h.at[pl.ds(tbase + ci * CH, CH)], srcb)
        pltpu.sync_copy(dst_h.at[pl.ds(tbase + ci * CH, CH)], dstb)

        @pl.loop(0, NB)
        def _batch(bi):
            off = bi * 16
            sa = srcb[pl.ds(off, 16)]
            da = dstb[pl.ds(off, 16)]
            s = jnp.where(fwd, sa, da)
            d = jnp.where(fwd, da, sa)
            as0 = plsc.load_gather(tbl_v, [s])
            as1 = plsc.load_gather(tbl_v, [s + NP])
            ad0 = plsc.load_gather(tbl_v, [d + 2 * NP])
            ad1 = plsc.load_gather(tbl_v, [d + 3 * NP])
            e0 = as0 + ad0
            e1 = as1 + ad1
            e0 = jnp.where(e0 >= 0, e0, 0.2 * e0)
            e1 = jnp.where(e1 >= 0, e1, 0.2 * e1)
            w0 = jnp.exp(e0)
            w1 = jnp.exp(e1)
            w0b[pl.ds(off, 16)] = w0
            w1b[pl.ds(off, 16)] = w1
            plsc.addupdate_scatter(den_v, [d], w0)
            plsc.addupdate_scatter(den_v, [d + NP], w1)

        pltpu.sync_copy(w0b, w0_out.at[pl.ds(ebase + ci * CH, CH)])
        pltpu.sync_copy(w1b, w1_out.at[pl.ds(ebase + ci * CH, CH)])

    pltpu.sync_copy(den_v, den_out.at[c * 16 + t])


def _logits(tbl2, src, dst, zden):
    mesh = plsc.VectorSubcoreMesh(core_axis_name="c", subcore_axis_name="s")
    f = pl.kernel(
        _logit_body,
        out_type=(jax.ShapeDtypeStruct((32, 2 * NP), jnp.float32),
                  jax.ShapeDtypeStruct((2 * E,), jnp.float32),
                  jax.ShapeDtypeStruct((2 * E,), jnp.float32)),
        mesh=mesh,
        compiler_params=pltpu.CompilerParams(needs_layout_passes=False),
        scratch_types=[
            pltpu.VMEM((4 * NP,), jnp.float32),
            pltpu.VMEM((2 * NP,), jnp.float32),
            pltpu.VMEM((CH,), jnp.int32),
            pltpu.VMEM((CH,), jnp.int32),
            pltpu.VMEM((CH,), jnp.float32),
            pltpu.VMEM((CH,), jnp.float32),
        ],
    )
    return f(tbl2, src, dst, zden)


NBUF = 5
CH_B2 = 4000
NB_B2 = CH_B2 // 16


def _scat_body(h2b, src_h, dst_h, w0_in, w1_in, z2, acc_out, acc_sh,
               srcb, dstb, w0b, w1b, gbufs, rowbufs, semg, sems):
    c = lax.axis_index("c")
    t = lax.axis_index("s")
    pltpu.sync_copy(z2, acc_sh.at[pl.ds(t * (NP // 16), NP // 16)])
    plsc.subcore_barrier()

    tbase = t * EDGES_PER_TILE_B
    ebase = c * E + tbase
    coff = c * NP
    fwd = c == 0
    himask = jnp.full((16,), -65536, jnp.int32)

    def issue_gather(m, b):
        sa = srcb[pl.ds(m * 16, 16)]
        da = dstb[pl.ds(m * 16, 16)]
        s = jnp.where(fwd, sa, da)
        return pltpu.async_copy(h2b.at[s + coff], gbufs[b], semg[b])

    @pl.loop(0, EDGES_PER_TILE_B // CH_B2)
    def _chunk(ci):
        pltpu.sync_copy(src_h.at[pl.ds(tbase + ci * CH_B2, CH_B2)], srcb)
        pltpu.sync_copy(dst_h.at[pl.ds(tbase + ci * CH_B2, CH_B2)], dstb)
        pltpu.sync_copy(w0_in.at[pl.ds(ebase + ci * CH_B2, CH_B2)], w0b)
        pltpu.sync_copy(w1_in.at[pl.ds(ebase + ci * CH_B2, CH_B2)], w1b)
        for b in range(NBUF):
            issue_gather(b, b)

        @pl.loop(0, NB_B2, step=NBUF)
        def _batch(bi):
            for b in range(NBUF):
                m = bi + b
                off = m * 16
                sa = dstb[pl.ds(off, 16)]
                sb = srcb[pl.ds(off, 16)]
                d = jnp.where(fwd, sa, sb)
                w0 = w0b[pl.ds(off, 16)]
                w1 = w1b[pl.ds(off, 16)]
                pltpu.make_async_copy(h2b.at[pl.ds(0, 16)], gbufs[b],
                                      semg[b]).wait()
                @pl.when(m >= NBUF)
                def _():
                    pltpu.make_async_copy(h2b.at[pl.ds(0, 16)], rowbufs[b],
                                          sems[b]).wait()
                gbuf = gbufs[b]
                rowbuf = rowbufs[b]
                for r in range(16):
                    w0r = w0[r]
                    w1r = w1[r]
                    for u in range(4):
                        v = gbuf[r, pl.ds(32 * u, 32)]
                        x = plsc.bitcast(v, jnp.int32)
                        lo = plsc.bitcast(x << 16, jnp.float32)
                        hi = plsc.bitcast(x & himask, jnp.float32)
                        rowbuf[r, pl.ds(16 * u, 16)] = lo * w0r
                        rowbuf[r, pl.ds(HD + 16 * u, 16)] = hi * w1r
                pltpu.async_copy(rowbufs[b], acc_sh.at[d], sems[b], add=True)

                @pl.when(m + NBUF < NB_B2)
                def _():
                    issue_gather(m + NBUF, b)

        for b in range(NBUF):
            pltpu.make_async_copy(h2b.at[pl.ds(0, 16)], rowbufs[b],
                                  sems[b]).wait()

    plsc.subcore_barrier()
    pltpu.sync_copy(acc_sh.at[pl.ds(t * (NP // 16), NP // 16)],
                    acc_out.at[c, pl.ds(t * (NP // 16), NP // 16)])


def _scatter(h2b, src, dst, w0, w1, z2):
    mesh = plsc.VectorSubcoreMesh(core_axis_name="c", subcore_axis_name="s")
    f = pl.kernel(
        _scat_body,
        out_type=jax.ShapeDtypeStruct((2, NP, D), jnp.float32),
        mesh=mesh,
        compiler_params=pltpu.CompilerParams(needs_layout_passes=False,
                                             use_tc_tiling_on_sc=False),
        scratch_types=[
            pltpu.VMEM_SHARED((NP, D), jnp.float32),
            pltpu.VMEM((CH_B2,), jnp.int32),
            pltpu.VMEM((CH_B2,), jnp.int32),
            pltpu.VMEM((CH_B2,), jnp.float32),
            pltpu.VMEM((CH_B2,), jnp.float32),
            [pltpu.VMEM((16, D), jnp.bfloat16)] * NBUF,
            [pltpu.VMEM((16, D), jnp.float32)] * NBUF,
            [pltpu.SemaphoreType.DMA] * NBUF,
            [pltpu.SemaphoreType.DMA] * NBUF,
        ],
    )
    return f(h2b, src, dst, w0, w1, z2)


def _mid_body(num_ref, den_ref, aa_ref, hf_ref, hu_ref, b2_ref, wfc_ref,
              bfc_ref, wo_ref, h3_ref):
    num = num_ref[...]
    den = jnp.sum(den_ref[...], axis=1)
    aa = aa_ref[...]

    def gat_out(k, h):
        el0 = aa[k, 0] + aa[k, 2]
        el1 = aa[k, 1] + aa[k, 3]
        wl0 = jnp.exp(jnp.where(el0 >= 0, el0, 0.2 * el0))
        wl1 = jnp.exp(jnp.where(el1 >= 0, el1, 0.2 * el1))
        n0 = num[k, :, :HD] + wl0[:, None] * h[:, :HD]
        n1 = num[k, :, HD:] + wl1[:, None] * h[:, HD:]
        d0 = den[k, 0] + wl0 + 1e-16
        d1 = den[k, 1] + wl1 + 1e-16
        out = jnp.concatenate([n0 / d0[:, None], n1 / d1[:, None]], axis=1)
        return jnp.maximum(out + b2_ref[k][None, :], 0.0)

    Hf = gat_out(0, hf_ref[...])
    Hu = gat_out(1, hu_ref[...])
    hcat = jnp.concatenate([Hf, Hu], axis=1)
    h = jnp.dot(hcat, wfc_ref[...], preferred_element_type=jnp.float32)
    h = jnp.maximum(h + bfc_ref[...], 0.0)
    h3_ref[...] = jnp.sum(h * wo_ref[...], axis=1)


def _mid(num2, den2, aa2, h2, b2, W_fc, bfc, wo_row):
    nblk = NP // BLK_C
    return pl.pallas_call(
        _mid_body,
        grid=(nblk,),
        in_specs=[
            pl.BlockSpec((2, BLK_C, D), lambda i: (0, i, 0)),
            pl.BlockSpec((2, 16, 2, BLK_C), lambda i: (0, 0, 0, i)),
            pl.BlockSpec((2, 8, BLK_C), lambda i: (0, 0, i)),
            pl.BlockSpec((BLK_C, D), lambda i: (i, 0)),
            pl.BlockSpec((BLK_C, D), lambda i: (i + nblk, 0)),
            pl.BlockSpec((2, D), lambda i: (0, 0)),
            pl.BlockSpec((2 * D, D), lambda i: (0, 0)),
            pl.BlockSpec((1, D), lambda i: (0, 0)),
            pl.BlockSpec((1, D), lambda i: (0, 0)),
        ],
        out_specs=pl.BlockSpec((BLK_C,), lambda i: (i,)),
        out_shape=jax.ShapeDtypeStruct((NP,), jnp.float32),
    )(num2, den2, aa2, h2, h2, b2, W_fc, bfc, wo_row)


def _out_body(h3, src, dst, zden, cvec, num_out, den_out, h3_v, num_v,
              den_v, srcb, dstb, cv, sem):
    c = lax.axis_index("c")
    t = lax.axis_index("s")
    pltpu.sync_copy(h3, h3_v)
    pltpu.sync_copy(zden.at[pl.ds(0, NP)], num_v)
    pltpu.sync_copy(zden.at[pl.ds(0, NP)], den_v)
    pltpu.sync_copy(cvec, cv)
    cvv = cv[...]
    cs = cvv[0]
    cd = cvv[1]
    ebase = (c * 16 + t) * EDGES_PER_TILE_D

    @pl.loop(0, EDGES_PER_TILE_D // CH)
    def _chunk(ci):
        pltpu.sync_copy(src.at[pl.ds(ebase + ci * CH, CH)], srcb)
        pltpu.sync_copy(dst.at[pl.ds(ebase + ci * CH, CH)], dstb)

        @pl.loop(0, NB)
        def _batch(bi):
            off = bi * 16
            s = srcb[pl.ds(off, 16)]
            d = dstb[pl.ds(off, 16)]
            hs = plsc.load_gather(h3_v, [s])
            hd = plsc.load_gather(h3_v, [d])
            e = cs * hs + cd * hd
            e = jnp.where(e >= 0, e, 0.2 * e)
            w = jnp.exp(e)
            plsc.addupdate_scatter(num_v, [d], w * hs)
            plsc.addupdate_scatter(den_v, [d], w)

    pltpu.sync_copy(num_v, num_out.at[c * 16 + t])
    pltpu.sync_copy(den_v, den_out.at[c * 16 + t])


def _conv_o(h3, src, dst, zden, cvec):
    mesh = plsc.VectorSubcoreMesh(core_axis_name="c", subcore_axis_name="s")
    f = pl.kernel(
        _out_body,
        out_type=(jax.ShapeDtypeStruct((32, NP), jnp.float32),
                  jax.ShapeDtypeStruct((32, NP), jnp.float32)),
        mesh=mesh,
        compiler_params=pltpu.CompilerParams(needs_layout_passes=False),
        scratch_types=[
            pltpu.VMEM((NP,), jnp.float32),
            pltpu.VMEM((NP,), jnp.float32),
            pltpu.VMEM((NP,), jnp.float32),
            pltpu.VMEM((CH,), jnp.int32),
            pltpu.VMEM((CH,), jnp.int32),
            pltpu.VMEM((16,), jnp.float32),
            pltpu.SemaphoreType.DMA,
        ],
    )
    return f(h3, src, dst, zden, cvec)


def _fin_body(h3_ref, num_ref, den_ref, scal_ref, out_ref):
    h3 = h3_ref[...]
    cs = scal_ref[0, 0]
    cd = scal_ref[0, 1]
    bo = scal_ref[0, 2]
    el = (cs + cd) * h3
    wl = jnp.exp(jnp.where(el >= 0, el, 0.2 * el))
    num = jnp.sum(num_ref[...], axis=0) + wl * h3
    den = jnp.sum(den_ref[...], axis=0) + wl + 1e-16
    out_ref[...] = jax.nn.sigmoid(num / den + bo)


def _fin(h3, num3, den3, scal):
    return pl.pallas_call(
        _fin_body,
        grid=(1,),
        in_specs=[
            pl.BlockSpec((NP,), lambda i: (0,)),
            pl.BlockSpec((32, NP), lambda i: (0, 0)),
            pl.BlockSpec((32, NP), lambda i: (0, 0)),
            pl.BlockSpec((1, 128), lambda i: (0, 0)),
        ],
        out_specs=pl.BlockSpec((NP,), lambda i: (0,)),
        out_shape=jax.ShapeDtypeStruct((NP,), jnp.float32),
    )(h3, num3, den3, scal)


def _pack_att(att_src, att_dst):
    a = jnp.zeros((D, 8), jnp.float32)
    a = a.at[:HD, 0].set(att_src[0])
    a = a.at[HD:, 1].set(att_src[1])
    a = a.at[:HD, 2].set(att_dst[0])
    a = a.at[HD:, 3].set(att_dst[1])
    return a


@jax.jit
def kernel(x, edge_index, W_f, att_src_f, att_dst_f, b_f,
           W_u, att_src_u, att_dst_u, b_u,
           W_fc, b_fc, W_o, att_src_o, att_dst_o, b_o):
    src = edge_index[0].astype(jnp.int32)
    dst = edge_index[1].astype(jnp.int32)
    x_p = jnp.pad(x, ((0, NP - N), (0, 0)))

    W2 = jnp.stack([W_f, W_u])
    perm = jnp.arange(D).reshape(2, HD).T.reshape(-1)
    W2p = W2[:, :, perm]
    A2 = jnp.stack([_pack_att(att_src_f, att_dst_f),
                    _pack_att(att_src_u, att_dst_u)])
    h2, h2b, aa2 = _proj(x_p, W2, W2p, A2)

    tbl2 = aa2[:, :4, :].reshape(-1)
    z2 = jnp.zeros((NP // 16, D), jnp.float32)
    zden = jnp.zeros((2 * NP,), jnp.float32)
    den, w0, w1 = _logits(tbl2, src, dst, zden)
    acc = _scatter(h2b, src, dst, w0, w1, z2)

    den2 = den.reshape(2, 16, 2, NP)
    b2 = jnp.stack([b_f, b_u])
    h3 = _mid(acc, den2, aa2, h2, b2, W_fc, b_fc.reshape(1, D),
              W_o.reshape(1, D))

    cvec = jnp.zeros((16,), jnp.float32)
    cvec = cvec.at[0].set(att_src_o[0, 0]).at[1].set(att_dst_o[0, 0])
    num3, den3 = _conv_o(h3, src, dst, zden, cvec)

    scal = jnp.zeros((1, 128), jnp.float32)
    scal = scal.at[0, 0].set(att_src_o[0, 0])
    scal = scal.at[0, 1].set(att_dst_o[0, 0])
    scal = scal.at[0, 2].set(b_o[0])
    num3, den3 = num3, den3
    out = _fin(h3, num3, den3, scal)
    return out[:N, None]
